# Initial kernel scaffold; baseline (speedup 1.0000x reference)
import jax
import jax.numpy as jnp
from jax.experimental import pallas as pl


def kernel(queries, neg_bank, pos_bank):
    raise NotImplementedError("write your pallas kernel here")


# speedup vs baseline: 2.8995x; 2.8995x over previous
import functools

import jax
import jax.numpy as jnp
from jax.experimental import pallas as pl
from jax.experimental.pallas import tpu as pltpu

ALPHA = 0.7
BETA = 0.3
EPS = 1e-12
BIG = 1e30


def _knn_min_kernel(nvalid_ref, q_ref, nb_ref, pb_ref, out_ref,
                    mneg_ref, mpos_ref):
    j = pl.program_id(1)
    nbj = pl.num_programs(1)
    bb = nb_ref.shape[1]

    @pl.when(j == 0)
    def _init():
        mneg_ref[:] = jnp.full(mneg_ref.shape, BIG, jnp.float32)
        mpos_ref[:] = jnp.full(mpos_ref.shape, BIG, jnp.float32)

    q = q_ref[:]
    nb = nb_ref[:]
    pb = pb_ref[:]

    dn = jax.lax.dot_general(q, nb, (((1,), (0,)), ((), ())),
                             preferred_element_type=jnp.float32)
    dp = jax.lax.dot_general(q, pb, (((1,), (0,)), ((), ())),
                             preferred_element_type=jnp.float32)
    nb2 = jnp.sum(nb * nb, axis=0, keepdims=True)
    pb2 = jnp.sum(pb * pb, axis=0, keepdims=True)

    col = j * bb + jax.lax.broadcasted_iota(jnp.int32, (1, bb), 1)
    valid = col < nvalid_ref[0]
    cn = jnp.where(valid, nb2 - 2.0 * dn, BIG)
    cp = jnp.where(valid, pb2 - 2.0 * dp, BIG)
    mneg_ref[:] = jnp.minimum(mneg_ref[:], jnp.min(cn, axis=1, keepdims=True))
    mpos_ref[:] = jnp.minimum(mpos_ref[:], jnp.min(cp, axis=1, keepdims=True))

    @pl.when(j == nbj - 1)
    def _fin():
        q2 = jnp.sum(q * q, axis=1, keepdims=True)
        dneg = jnp.sqrt(jnp.maximum(q2 + mneg_ref[:], 0.0) + EPS)
        dpos = jnp.sqrt(jnp.maximum(q2 + mpos_ref[:], 0.0) + EPS)
        out_ref[:] = ALPHA * dneg - BETA * dpos


@functools.partial(jax.jit, static_argnames=("qb", "bb"))
def _run(queries, neg_bank, pos_bank, qb, bb):
    nq, d = queries.shape
    n = neg_bank.shape[0]
    n_pad = ((n + bb - 1) // bb) * bb
    nbt = jnp.pad(neg_bank.T, ((0, 0), (0, n_pad - n)))
    pbt = jnp.pad(pos_bank.T, ((0, 0), (0, n_pad - n)))
    nvalid = jnp.full((1,), n, jnp.int32)
    grid = (nq // qb, n_pad // bb)
    out = pl.pallas_call(
        _knn_min_kernel,
        grid=grid,
        in_specs=[
            pl.BlockSpec(memory_space=pltpu.SMEM),
            pl.BlockSpec((qb, d), lambda i, j: (i, 0)),
            pl.BlockSpec((d, bb), lambda i, j: (0, j)),
            pl.BlockSpec((d, bb), lambda i, j: (0, j)),
        ],
        out_specs=pl.BlockSpec((qb, 1), lambda i, j: (i, 0)),
        out_shape=jax.ShapeDtypeStruct((nq, 1), jnp.float32),
        scratch_shapes=[
            pltpu.VMEM((qb, 1), jnp.float32),
            pltpu.VMEM((qb, 1), jnp.float32),
        ],
    )(nvalid, queries, nbt, pbt)
    return out[:, 0]


def kernel(queries, neg_bank, pos_bank):
    return _run(queries, neg_bank, pos_bank, qb=448, bb=1024)


# speedup vs baseline: 4.0410x; 1.3937x over previous
import functools

import jax
import jax.numpy as jnp
from jax.experimental import pallas as pl
from jax.experimental.pallas import tpu as pltpu

ALPHA = 0.7
BETA = 0.3
EPS = 1e-12
BIG = 1e30


def _knn_min_kernel(qt_ref, nb_ref, pb_ref, out_ref, mneg_ref, mpos_ref):
    j = pl.program_id(0)
    i = pl.program_id(1)
    nbj = pl.num_programs(0)
    qb = qt_ref.shape[1]

    qt = qt_ref[:]
    nb = nb_ref[:]
    pb = pb_ref[:]

    dn = jax.lax.dot_general(nb, qt, (((1,), (0,)), ((), ())),
                             preferred_element_type=jnp.float32)
    dp = jax.lax.dot_general(pb, qt, (((1,), (0,)), ((), ())),
                             preferred_element_type=jnp.float32)
    nb2 = jnp.sum(nb * nb, axis=1, keepdims=True)
    pb2 = jnp.sum(pb * pb, axis=1, keepdims=True)

    mn = jnp.min(nb2 - 2.0 * dn, axis=0, keepdims=True)
    mp = jnp.min(pb2 - 2.0 * dp, axis=0, keepdims=True)

    sl = pl.ds(i * qb, qb)
    prev_n = jnp.where(j == 0, BIG, mneg_ref[:, sl])
    prev_p = jnp.where(j == 0, BIG, mpos_ref[:, sl])
    acc_n = jnp.minimum(prev_n, mn)
    acc_p = jnp.minimum(prev_p, mp)
    mneg_ref[:, sl] = acc_n
    mpos_ref[:, sl] = acc_p

    @pl.when(j == nbj - 1)
    def _fin():
        q2 = jnp.sum(qt * qt, axis=0, keepdims=True)
        dneg = jnp.sqrt(jnp.maximum(q2 + acc_n, 0.0) + EPS)
        dpos = jnp.sqrt(jnp.maximum(q2 + acc_p, 0.0) + EPS)
        out_ref[:] = ALPHA * dneg - BETA * dpos


@functools.partial(jax.jit, static_argnames=("qb", "bb"))
def _run(queries, neg_bank, pos_bank, qb, bb):
    nq, d = queries.shape
    n = neg_bank.shape[0]
    qt = queries.T
    grid = (n // bb, nq // qb)
    out = pl.pallas_call(
        _knn_min_kernel,
        grid=grid,
        in_specs=[
            pl.BlockSpec((d, qb), lambda j, i: (0, i)),
            pl.BlockSpec((bb, d), lambda j, i: (j, 0)),
            pl.BlockSpec((bb, d), lambda j, i: (j, 0)),
        ],
        out_specs=pl.BlockSpec((1, qb), lambda j, i: (0, i)),
        out_shape=jax.ShapeDtypeStruct((1, nq), jnp.float32),
        scratch_shapes=[
            pltpu.VMEM((1, nq), jnp.float32),
            pltpu.VMEM((1, nq), jnp.float32),
        ],
    )(qt, neg_bank, pos_bank)
    return out[0]


def kernel(queries, neg_bank, pos_bank):
    return _run(queries, neg_bank, pos_bank, qb=896, bb=1000)
